# Initial kernel scaffold; baseline (speedup 1.0000x reference)
import jax
import jax.numpy as jnp
from jax.experimental import pallas as pl


def kernel(h, edge_index, r, norm, W_basis0, comb0, W_loop0, bias0, W_basis1, comb1, W_loop1, bias1):
    raise NotImplementedError("write your pallas kernel here")


# speedup vs baseline: 6.8365x; 6.8365x over previous
import functools

import jax
import jax.numpy as jnp
from jax import lax
from jax.experimental import pallas as pl
from jax.experimental.pallas import tpu as pltpu
from jax.experimental.pallas import tpu_sc as plsc

N = 10000
E = 320000
D = 128
R = 16
B = 4

NC = 2
NS = 16
NW = NC * NS
C = 128
NCH = -(-E // (NW * C))
EPW = NCH * C
E_PAD = EPW * NW
N_PAD = 10240
NBLK = 1280
NB = N_PAD // NBLK


def _table_body(comb_ref, wb_ref, x_ref, o_ref):
    r_id = pl.program_id(0)
    w = (comb_ref[r_id, 0] * wb_ref[0]
         + comb_ref[r_id, 1] * wb_ref[1]
         + comb_ref[r_id, 2] * wb_ref[2]
         + comb_ref[r_id, 3] * wb_ref[3])
    o_ref[0] = jnp.dot(x_ref[...], w, preferred_element_type=jnp.float32)


def _make_table(comb, wb, x_pad):
    return pl.pallas_call(
        _table_body,
        grid=(R, NB),
        in_specs=[
            pl.BlockSpec(memory_space=pltpu.SMEM),
            pl.BlockSpec((B, D, D), lambda r, i: (0, 0, 0)),
            pl.BlockSpec((NBLK, D), lambda r, i: (i, 0)),
        ],
        out_specs=pl.BlockSpec((1, NBLK, D), lambda r, i: (r, i, 0)),
        out_shape=jax.ShapeDtypeStruct((R, N_PAD, D), jnp.float32),
    )(comb, wb, x_pad)


def _combine_body(a0_ref, a1_ref, x_ref, wl_ref, b_ref, o_ref):
    acc = (a0_ref[...] + a1_ref[...]
           + jnp.dot(x_ref[...], wl_ref[...], preferred_element_type=jnp.float32)
           + b_ref[...])
    o_ref[...] = jnp.maximum(acc, 0.0)


def _combine(a0, a1, x_pad, wl, bias2d):
    return pl.pallas_call(
        _combine_body,
        grid=(NB,),
        in_specs=[
            pl.BlockSpec((NBLK, D), lambda i: (i, 0)),
            pl.BlockSpec((NBLK, D), lambda i: (i, 0)),
            pl.BlockSpec((NBLK, D), lambda i: (i, 0)),
            pl.BlockSpec((D, D), lambda i: (0, 0)),
            pl.BlockSpec((1, D), lambda i: (0, 0)),
        ],
        out_specs=pl.BlockSpec((NBLK, D), lambda i: (i, 0)),
        out_shape=jax.ShapeDtypeStruct((N_PAD, D), jnp.float32),
    )(a0, a1, x_pad, wl, bias2d)


_SPLAT_DN = lax.GatherDimensionNumbers(
    offset_dims=(), collapsed_slice_dims=(0,), start_index_map=(0,))


def _splat16(vec, l):
    idx = jnp.full((16, 1), l, jnp.int32)
    return lax.gather(vec, idx, _SPLAT_DN, (1,),
                      mode=lax.GatherScatterMode.PROMISE_IN_BOUNDS)


def _sc_body(table_hbm, gidx_hbm, dst_hbm, norm_hbm, out_hbm,
             norm_v, gidx_v, dst_v, rows_v, agg, sem):
    c = lax.axis_index("c")
    s = lax.axis_index("s")
    wid = c * NS + s

    pltpu.sync_copy(gidx_hbm.at[wid], gidx_v)
    pltpu.sync_copy(norm_hbm.at[wid], norm_v)
    pltpu.sync_copy(dst_hbm.at[wid], dst_v)

    def _zrow(i, carry):
        for j in range(8):
            rows_v[i, pl.ds(j * 16, 16)] = jnp.zeros((16,), jnp.float32)
        return carry
    lax.fori_loop(0, C, _zrow, 0)
    stripe = N_PAD // NS
    for k in range(stripe // C):
        pltpu.sync_copy(rows_v, agg.at[pl.ds(s * stripe + k * C, C)])

    plsc.subcore_barrier()

    def _chunk(ch, carry):
        gsl = gidx_v.at[pl.ds(ch * C, C)]
        pltpu.async_copy(table_hbm.at[gsl], rows_v, sem).wait()

        def _scale(g, carry2):
            nvec = norm_v[pl.ds(ch * C + g * 16, 16)]
            for l in range(16):
                nb = _splat16(nvec, l)
                i = g * 16 + l
                for j in range(8):
                    sl = pl.ds(j * 16, 16)
                    rows_v[i, sl] = rows_v[i, sl] * nb
            return carry2
        lax.fori_loop(0, C // 16, _scale, 0)

        pltpu.sync_copy(rows_v, agg.at[dst_v.at[ch]], add=True)
        return carry
    lax.fori_loop(0, NCH, _chunk, 0)

    plsc.subcore_barrier()

    pltpu.sync_copy(agg.at[pl.ds(s * stripe, stripe)],
                    out_hbm.at[c, pl.ds(s * stripe, stripe)])


_sc_call = functools.partial(
    pl.kernel,
    out_type=jax.ShapeDtypeStruct((NC, N_PAD, D), jnp.float32),
    mesh=plsc.VectorSubcoreMesh(core_axis_name="c", subcore_axis_name="s",
                                num_cores=NC, num_subcores=NS),
    scratch_types=[
        pltpu.VMEM((EPW,), jnp.float32),
        pltpu.VMEM((EPW,), jnp.int32),
        pltpu.VMEM((NCH, C), jnp.int32),
        pltpu.VMEM((C, D), jnp.float32),
        pltpu.VMEM_SHARED((N_PAD, D), jnp.float32),
        pltpu.SemaphoreType.DMA,
    ],
)(_sc_body)


def _layer(x_pad, wb, comb, wl, bias, gidx_w, dst_w, norm_w):
    table = _make_table(comb, wb, x_pad)
    parts = _sc_call(table.reshape(R * N_PAD, D), gidx_w, dst_w, norm_w)
    return _combine(parts[0], parts[1], x_pad, wl, bias.reshape(1, D))


def kernel(h, edge_index, r, norm,
           W_basis0, comb0, W_loop0, bias0,
           W_basis1, comb1, W_loop1, bias1):
    src = edge_index[0]
    dst = edge_index[1]
    pad = E_PAD - E
    gidx_w = jnp.pad(r * N_PAD + src, (0, pad)).reshape(NW, EPW)
    dst_w = jnp.pad(dst, (0, pad)).reshape(NW, NCH, C)
    norm_w = jnp.pad(norm[:, 0], (0, pad)).reshape(NW, EPW)
    x0 = jnp.pad(h, ((0, N_PAD - N), (0, 0)))

    h1 = _layer(x0, W_basis0, comb0, W_loop0, bias0, gidx_w, dst_w, norm_w)
    h2 = _layer(h1, W_basis1, comb1, W_loop1, bias1, gidx_w, dst_w, norm_w)
    return h2[:N]
